# baseline (speedup 1.0000x reference)
import jax
import jax.numpy as jnp
from jax import lax
from jax.experimental import pallas as pl
from jax.experimental.pallas import tpu as pltpu
from jax.experimental.pallas import tpu_sc as plsc

_LANES = 16


def kernel(word_embeddings, pos_table):
    B, S, D = word_embeddings.shape
    R = 16
    we2 = word_embeddings.reshape(B * S, D)
    mesh = plsc.VectorSubcoreMesh(core_axis_name="c", subcore_axis_name="s")

    @pl.kernel(
        out_type=jax.ShapeDtypeStruct((B * S, D), word_embeddings.dtype),
        mesh=mesh,
    )
    def sc_kernel(we_hbm, pos_hbm, o_hbm):
        def body(we_vmem, pos_vmem, o_vmem):
            @pl.loop(0, R)
            def _(r):
                @pl.loop(0, D, step=_LANES)
                def _(c):
                    slc = (pl.ds(r, 1), pl.ds(c, _LANES))
                    o_vmem.at[*slc][...] = (
                        we_vmem.at[*slc][...] + pos_vmem.at[*slc][...]
                    )

        pltpu.emit_pipeline(
            body,
            grid=((B * S) // R,),
            in_specs=[
                pl.BlockSpec((R, D), index_map=lambda i: (i, 0)),
                pl.BlockSpec((R, D), index_map=lambda i: (lax.rem(i, S // R), 0)),
            ],
            out_specs=[pl.BlockSpec((R, D), index_map=lambda i: (i, 0))],
            core_axis_name=("c", "s"),
            dimension_semantics=(pltpu.PARALLEL,),
        )(we_hbm, pos_hbm, o_hbm)

    return sc_kernel(we2, pos_table).reshape(B, S, D)


# speedup vs baseline: 4.3237x; 4.3237x over previous
import jax
import jax.numpy as jnp
from jax.experimental import pallas as pl


def _add_kernel(we_ref, pos_ref, out_ref):
    out_ref[...] = we_ref[...] + pos_ref[...][None, :, :]


def kernel(word_embeddings, pos_table):
    B, S, D = word_embeddings.shape
    BS = 512
    grid = (S // BS,)
    return pl.pallas_call(
        _add_kernel,
        grid=grid,
        in_specs=[
            pl.BlockSpec((B, BS, D), lambda s: (0, s, 0)),
            pl.BlockSpec((BS, D), lambda s: (s, 0)),
        ],
        out_specs=pl.BlockSpec((B, BS, D), lambda s: (0, s, 0)),
        out_shape=jax.ShapeDtypeStruct((B, S, D), word_embeddings.dtype),
    )(word_embeddings, pos_table)


# speedup vs baseline: 4.3564x; 1.0076x over previous
import jax
import jax.numpy as jnp
from jax.experimental import pallas as pl


def _add_kernel(we_ref, pos_ref, out_ref):
    out_ref[...] = we_ref[...] + pos_ref[...][None, :, :]


def kernel(word_embeddings, pos_table):
    B, S, D = word_embeddings.shape
    BS = 2048
    grid = (S // BS, B)
    return pl.pallas_call(
        _add_kernel,
        grid=grid,
        in_specs=[
            pl.BlockSpec((1, BS, D), lambda s, b: (b, s, 0)),
            pl.BlockSpec((BS, D), lambda s, b: (s, 0)),
        ],
        out_specs=pl.BlockSpec((1, BS, D), lambda s, b: (b, s, 0)),
        out_shape=jax.ShapeDtypeStruct((B, S, D), word_embeddings.dtype),
    )(word_embeddings, pos_table)
